# Initial kernel scaffold; baseline (speedup 1.0000x reference)
import jax
import jax.numpy as jnp
from jax.experimental import pallas as pl


def kernel(observed, modeled):
    raise NotImplementedError("write your pallas kernel here")


# speedup vs baseline: 8.9102x; 8.9102x over previous
import functools

import jax
import jax.numpy as jnp
from jax.experimental import pallas as pl
from jax.experimental.pallas import tpu as pltpu

EPS_ = 1e-6
B_, S_, D_ = 8, 512, 64
NEG_BIG = -3.0e38
POS_BIG = 3.0e38


def _shift_right(x, s, fill):
    return jnp.concatenate(
        [jnp.full((x.shape[0], s), fill, x.dtype), x[:, :-s]], axis=1
    )


def _cumsum_lanes(x):
    s = 1
    while s < x.shape[1]:
        x = x + _shift_right(x, s, 0.0)
        s *= 2
    return x


def _cummin_lanes(x):
    s = 1
    while s < x.shape[1]:
        x = jnp.minimum(x, _shift_right(x, s, POS_BIG))
        s *= 2
    return x


def _dtw_body(o_ref, m_ref, out_ref, d_ref):
    for b in range(B_):
        o_b = o_ref[b]
        m_b = m_ref[b]
        dot = jax.lax.dot_general(
            o_b, m_b, (((1,), (1,)), ((), ())),
            preferred_element_type=jnp.float32,
        )
        sq_o = jnp.sum(o_b * o_b, axis=1)[:, None]
        sq_m = jnp.sum(m_b * m_b, axis=1)[None, :]
        sum_o = jnp.sum(o_b, axis=1)[:, None]
        sum_m = jnp.sum(m_b, axis=1)[None, :]
        dist2 = (sq_o + sq_m - 2.0 * dot
                 + (2.0 * EPS_) * (sum_o - sum_m)
                 + D_ * EPS_ * EPS_)
        cost = jnp.sqrt(jnp.maximum(dist2, 0.0))
        d_ref[:, b, :] = cost

    prev0 = _cumsum_lanes(d_ref[0])

    def step(i, prev):
        d_i = d_ref[i]
        r_shift = _shift_right(prev, 1, POS_BIG)
        a = d_i + jnp.minimum(prev, r_shift)
        c = _cumsum_lanes(d_i)
        cur = c + _cummin_lanes(a - c)
        return cur

    last = jax.lax.fori_loop(1, S_, step, prev0)

    loss = jnp.sum(last[:, S_ - 1]) * (1.0 / B_)
    out_ref[...] = jnp.full((1, 1), loss, jnp.float32)


@jax.jit
def _dtw_loss(observed, modeled):
    out = pl.pallas_call(
        _dtw_body,
        out_shape=jax.ShapeDtypeStruct((1, 1), jnp.float32),
        in_specs=[
            pl.BlockSpec(memory_space=pltpu.VMEM),
            pl.BlockSpec(memory_space=pltpu.VMEM),
        ],
        out_specs=pl.BlockSpec(memory_space=pltpu.VMEM),
        scratch_shapes=[pltpu.VMEM((S_, B_, S_), jnp.float32)],
    )(observed, modeled)
    return out[0, 0]


def kernel(observed, modeled):
    return _dtw_loss(observed, modeled)


# speedup vs baseline: 15.0738x; 1.6918x over previous
import jax
import jax.numpy as jnp
from jax.experimental import pallas as pl
from jax.experimental.pallas import tpu as pltpu

EPS_ = 1e-6
B_, S_, D_ = 8, 512, 64
POS_BIG = 3.0e38


def _shift_right(x, s, fill):
    return jnp.concatenate(
        [jnp.full((x.shape[0], s), fill, x.dtype), x[:, :-s]], axis=1
    )


def _cummin_lanes(x):
    s = 1
    while s < x.shape[1]:
        x = jnp.minimum(x, _shift_right(x, s, POS_BIG))
        s *= 2
    return x


def _dtw_body(o_ref, m_ref, out_ref, c_ref, e_ref):
    rid = jax.lax.broadcasted_iota(jnp.int32, (S_, S_), 0)
    cid = jax.lax.broadcasted_iota(jnp.int32, (S_, S_), 1)
    u_strict = jnp.where(rid < cid, 1.0, 0.0).astype(jnp.float32)

    for b in range(B_):
        o_b = o_ref[b]
        m_b = m_ref[b]
        dot = jax.lax.dot_general(
            o_b, m_b, (((1,), (1,)), ((), ())),
            preferred_element_type=jnp.float32,
        )
        sq_o = jnp.sum(o_b * o_b, axis=1)[:, None]
        sq_m = jnp.sum(m_b * m_b, axis=1)[None, :]
        sum_o = jnp.sum(o_b, axis=1)[:, None]
        sum_m = jnp.sum(m_b, axis=1)[None, :]
        dist2 = (sq_o + sq_m - 2.0 * dot
                 + (2.0 * EPS_) * (sum_o - sum_m)
                 + D_ * EPS_ * EPS_)
        cost = jnp.sqrt(jnp.maximum(dist2, 0.0))
        e = jax.lax.dot_general(
            cost, u_strict, (((1,), (0,)), ((), ())),
            preferred_element_type=jnp.float32,
        )
        c = e + cost
        c_ref[:, b, :] = c
        e_ref[:, b, :] = e

    prev0 = c_ref[0]

    def step(i, prev):
        c_i = c_ref[i]
        e_i = e_ref[i]
        t = _shift_right(prev - c_i, 1, POS_BIG)
        x = jnp.minimum(prev - e_i, t)
        return c_i + _cummin_lanes(x)

    last = jax.lax.fori_loop(1, S_, step, prev0)

    loss = jnp.sum(last[:, S_ - 1]) * (1.0 / B_)
    out_ref[...] = jnp.full((1, 1), loss, jnp.float32)


@jax.jit
def _dtw_loss(observed, modeled):
    out = pl.pallas_call(
        _dtw_body,
        out_shape=jax.ShapeDtypeStruct((1, 1), jnp.float32),
        in_specs=[
            pl.BlockSpec(memory_space=pltpu.VMEM),
            pl.BlockSpec(memory_space=pltpu.VMEM),
        ],
        out_specs=pl.BlockSpec(memory_space=pltpu.VMEM),
        scratch_shapes=[
            pltpu.VMEM((S_, B_, S_), jnp.float32),
            pltpu.VMEM((S_, B_, S_), jnp.float32),
        ],
    )(observed, modeled)
    return out[0, 0]


def kernel(observed, modeled):
    return _dtw_loss(observed, modeled)


# speedup vs baseline: 21.2389x; 1.4090x over previous
import jax
import jax.numpy as jnp
from jax.experimental import pallas as pl
from jax.experimental.pallas import tpu as pltpu

EPS_ = 1e-6
B_, S_, D_ = 8, 512, 64
POS_BIG = 3.0e38


def _shift_right(x, s, fill):
    return jnp.concatenate(
        [jnp.full((x.shape[0], s), fill, x.dtype), x[:, :-s]], axis=1
    )


def _tree_min(vals):
    while len(vals) > 1:
        nxt = [jnp.minimum(vals[i], vals[i + 1])
               for i in range(0, len(vals) - 1, 2)]
        if len(vals) % 2:
            nxt.append(vals[-1])
        vals = nxt
    return vals[0]


R_ = 23


def _local_scan(x):
    return _tree_min([x] + [_shift_right(x, s, POS_BIG) for s in range(1, R_)])


def _dtw_body(o_ref, m_ref, out_ref, c_ref, e_ref):
    rid = jax.lax.broadcasted_iota(jnp.int32, (S_, S_), 0)
    cid = jax.lax.broadcasted_iota(jnp.int32, (S_, S_), 1)
    u_strict = jnp.where(rid < cid, 1.0, 0.0).astype(jnp.float32)

    for b in range(B_):
        o_b = o_ref[b]
        m_b = m_ref[b]
        dot = jax.lax.dot_general(
            o_b, m_b, (((1,), (1,)), ((), ())),
            preferred_element_type=jnp.float32,
        )
        sq_o = jnp.sum(o_b * o_b, axis=1)[:, None]
        sq_m = jnp.sum(m_b * m_b, axis=1)[None, :]
        sum_o = jnp.sum(o_b, axis=1)[:, None]
        sum_m = jnp.sum(m_b, axis=1)[None, :]
        dist2 = (sq_o + sq_m - 2.0 * dot
                 + (2.0 * EPS_) * (sum_o - sum_m)
                 + D_ * EPS_ * EPS_)
        cost = jnp.sqrt(jnp.maximum(dist2, 0.0))
        e = jax.lax.dot_general(
            cost, u_strict, (((1,), (0,)), ((), ())),
            preferred_element_type=jnp.float32,
        )
        c = e + cost
        c_ref[:, b, :] = c
        e_ref[:, b, :] = e

    prev0 = c_ref[0]

    def step(i, prev):
        c_i = c_ref[i]
        e_i = e_ref[i]
        w = prev - e_i
        t = prev - c_i
        aw = _local_scan(w)
        at = _local_scan(t)
        parts = ([aw] + [_shift_right(aw, R_ * k, POS_BIG)
                         for k in range(1, R_)]
                 + [_shift_right(at, 1 + R_ * k, POS_BIG)
                    for k in range(R_)])
        return c_i + _tree_min(parts)

    last = jax.lax.fori_loop(1, S_, step, prev0)

    loss = jnp.sum(last[:, S_ - 1]) * (1.0 / B_)
    out_ref[...] = jnp.full((1, 1), loss, jnp.float32)


@jax.jit
def _dtw_loss(observed, modeled):
    out = pl.pallas_call(
        _dtw_body,
        out_shape=jax.ShapeDtypeStruct((1, 1), jnp.float32),
        in_specs=[
            pl.BlockSpec(memory_space=pltpu.VMEM),
            pl.BlockSpec(memory_space=pltpu.VMEM),
        ],
        out_specs=pl.BlockSpec(memory_space=pltpu.VMEM),
        scratch_shapes=[
            pltpu.VMEM((S_, B_, S_), jnp.float32),
            pltpu.VMEM((S_, B_, S_), jnp.float32),
        ],
    )(observed, modeled)
    return out[0, 0]


def kernel(observed, modeled):
    return _dtw_loss(observed, modeled)


# speedup vs baseline: 27.0945x; 1.2757x over previous
import jax
import jax.numpy as jnp
from jax.experimental import pallas as pl
from jax.experimental.pallas import tpu as pltpu

EPS_ = 1e-6
B_, S_, D_ = 8, 512, 64
POS_BIG = 3.0e38


def _shift_right(x, s, fill):
    return jnp.concatenate(
        [jnp.full((x.shape[0], s), fill, x.dtype), x[:, :-s]], axis=1
    )


def _tree_min(vals):
    while len(vals) > 1:
        nxt = [jnp.minimum(vals[i], vals[i + 1])
               for i in range(0, len(vals) - 1, 2)]
        if len(vals) % 2:
            nxt.append(vals[-1])
        vals = nxt
    return vals[0]


R_ = 23


def _local_scan(x):
    return _tree_min([x] + [_shift_right(x, s, POS_BIG) for s in range(1, R_)])


def _dtw_body(o_ref, m_ref, out_ref, c_ref, e_ref):
    rid = jax.lax.broadcasted_iota(jnp.int32, (S_, S_), 0)
    cid = jax.lax.broadcasted_iota(jnp.int32, (S_, S_), 1)
    u_strict = jnp.where(rid < cid, 1.0, 0.0).astype(jnp.float32)

    for b in range(B_):
        o_b = o_ref[b]
        m_b = m_ref[b]
        dot = jax.lax.dot_general(
            o_b, m_b, (((1,), (1,)), ((), ())),
            preferred_element_type=jnp.float32,
        )
        sq_o = jnp.sum(o_b * o_b, axis=1)[:, None]
        sq_m = jnp.sum(m_b * m_b, axis=1)[None, :]
        sum_o = jnp.sum(o_b, axis=1)[:, None]
        sum_m = jnp.sum(m_b, axis=1)[None, :]
        dist2 = (sq_o + sq_m - 2.0 * dot
                 + (2.0 * EPS_) * (sum_o - sum_m)
                 + D_ * EPS_ * EPS_)
        cost = jnp.sqrt(jnp.maximum(dist2, 0.0))
        e = jax.lax.dot_general(
            cost, u_strict, (((1,), (0,)), ((), ())),
            preferred_element_type=jnp.float32,
        )
        c = e + cost
        c_ref[:, b, :] = c
        e_ref[:, b, :] = e

    prev0 = c_ref[0]

    def step(i, prev):
        c_i = c_ref[i]
        e_i = e_ref[i]
        t = prev - c_i
        w = prev - e_i
        at = _local_scan(t)
        parts = [_shift_right(at, 1 + R_ * k, POS_BIG)
                 for k in range(R_)]
        return c_i + jnp.minimum(w, _tree_min(parts))

    last = jax.lax.fori_loop(1, S_, step, prev0)

    loss = jnp.sum(last[:, S_ - 1]) * (1.0 / B_)
    out_ref[...] = jnp.full((1, 1), loss, jnp.float32)


@jax.jit
def _dtw_loss(observed, modeled):
    out = pl.pallas_call(
        _dtw_body,
        out_shape=jax.ShapeDtypeStruct((1, 1), jnp.float32),
        in_specs=[
            pl.BlockSpec(memory_space=pltpu.VMEM),
            pl.BlockSpec(memory_space=pltpu.VMEM),
        ],
        out_specs=pl.BlockSpec(memory_space=pltpu.VMEM),
        scratch_shapes=[
            pltpu.VMEM((S_, B_, S_), jnp.float32),
            pltpu.VMEM((S_, B_, S_), jnp.float32),
        ],
    )(observed, modeled)
    return out[0, 0]


def kernel(observed, modeled):
    return _dtw_loss(observed, modeled)


# speedup vs baseline: 31.4388x; 1.1603x over previous
import jax
import jax.numpy as jnp
from jax.experimental import pallas as pl
from jax.experimental.pallas import tpu as pltpu

EPS_ = 1e-6
B_, S_, D_ = 8, 512, 64
POS_BIG = 3.0e38


def _shift_right(x, s, fill):
    return jnp.concatenate(
        [jnp.full((x.shape[0], s), fill, x.dtype), x[:, :-s]], axis=1
    )


def _tree_min(vals):
    while len(vals) > 1:
        nxt = [jnp.minimum(vals[i], vals[i + 1])
               for i in range(0, len(vals) - 1, 2)]
        if len(vals) % 2:
            nxt.append(vals[-1])
        vals = nxt
    return vals[0]


R_ = 23
HL_ = 256
RH_ = 16


def _local_scan(x):
    return _tree_min([x] + [_shift_right(x, s, POS_BIG) for s in range(1, R_)])


def _dtw_body(o_ref, m_ref, out_ref, c_ref, e_ref):
    rid = jax.lax.broadcasted_iota(jnp.int32, (S_, S_), 0)
    cid = jax.lax.broadcasted_iota(jnp.int32, (S_, S_), 1)
    u_strict = jnp.where(rid < cid, 1.0, 0.0).astype(jnp.float32)

    for b in range(B_):
        o_b = o_ref[b]
        m_b = m_ref[b]
        dot = jax.lax.dot_general(
            o_b, m_b, (((1,), (1,)), ((), ())),
            preferred_element_type=jnp.float32,
        )
        sq_o = jnp.sum(o_b * o_b, axis=1)[:, None]
        sq_m = jnp.sum(m_b * m_b, axis=1)[None, :]
        sum_o = jnp.sum(o_b, axis=1)[:, None]
        sum_m = jnp.sum(m_b, axis=1)[None, :]
        dist2 = (sq_o + sq_m - 2.0 * dot
                 + (2.0 * EPS_) * (sum_o - sum_m)
                 + D_ * EPS_ * EPS_)
        cost = jnp.sqrt(jnp.maximum(dist2, 0.0))
        e = jax.lax.dot_general(
            cost, u_strict, (((1,), (0,)), ((), ())),
            preferred_element_type=jnp.float32,
        )
        c = e + cost
        c_ref[:, b, :] = c
        e_ref[:, b, :] = e

    prev0 = c_ref[0]

    def _half_update(prev, c_h, e_h, extra):
        t = prev - c_h
        w = prev - e_h
        at = _tree_min([t] + [_shift_right(t, s, POS_BIG)
                              for s in range(1, RH_)])
        parts = [_shift_right(at, 1 + RH_ * k, POS_BIG)
                 for k in range(RH_)]
        m = _tree_min(parts)
        if extra is not None:
            m = jnp.minimum(m, extra)
        mint = jnp.min(t, axis=1, keepdims=True)
        return c_h + jnp.minimum(w, m), mint

    l1, ml1 = _half_update(prev0[:, :HL_], c_ref[1][:, :HL_],
                           e_ref[1][:, :HL_], None)

    def step(k, carry):
        pl, pr, ml = carry
        c_k = c_ref[k]
        c_p = c_ref[k - 1]
        e_k = e_ref[k]
        e_p = e_ref[k - 1]
        nl, nml = _half_update(pl, c_k[:, :HL_], e_k[:, :HL_], None)
        nr, _ = _half_update(pr, c_p[:, HL_:], e_p[:, HL_:], ml)
        return nl, nr, nml

    ll, lr, lml = jax.lax.fori_loop(2, S_, step,
                                    (l1, prev0[:, HL_:], ml1))
    last_r, _ = _half_update(lr, c_ref[S_ - 1][:, HL_:],
                             e_ref[S_ - 1][:, HL_:], lml)
    last = last_r

    loss = jnp.sum(last[:, HL_ - 1]) * (1.0 / B_)
    out_ref[...] = jnp.full((1, 1), loss, jnp.float32)


@jax.jit
def _dtw_loss(observed, modeled):
    out = pl.pallas_call(
        _dtw_body,
        out_shape=jax.ShapeDtypeStruct((1, 1), jnp.float32),
        in_specs=[
            pl.BlockSpec(memory_space=pltpu.VMEM),
            pl.BlockSpec(memory_space=pltpu.VMEM),
        ],
        out_specs=pl.BlockSpec(memory_space=pltpu.VMEM),
        scratch_shapes=[
            pltpu.VMEM((S_, B_, S_), jnp.float32),
            pltpu.VMEM((S_, B_, S_), jnp.float32),
        ],
    )(observed, modeled)
    return out[0, 0]


def kernel(observed, modeled):
    return _dtw_loss(observed, modeled)


# speedup vs baseline: 33.1893x; 1.0557x over previous
import jax
import jax.numpy as jnp
from jax.experimental import pallas as pl
from jax.experimental.pallas import tpu as pltpu

EPS_ = 1e-6
B_, S_, D_ = 8, 512, 64
POS_BIG = 3.0e38


def _shift_right(x, s, fill):
    return jnp.concatenate(
        [jnp.full((x.shape[0], s), fill, x.dtype), x[:, :-s]], axis=1
    )


def _tree_min(vals):
    while len(vals) > 1:
        nxt = [jnp.minimum(vals[i], vals[i + 1])
               for i in range(0, len(vals) - 1, 2)]
        if len(vals) % 2:
            nxt.append(vals[-1])
        vals = nxt
    return vals[0]


CH_ = 128
NCH_ = 4
RC_ = 12


def _local_scan(x):
    return _tree_min([x] + [_shift_right(x, s, POS_BIG) for s in range(1, R_)])


def _dtw_body(o_ref, m_ref, out_ref, c_ref, e_ref):
    rid = jax.lax.broadcasted_iota(jnp.int32, (S_, S_), 0)
    cid = jax.lax.broadcasted_iota(jnp.int32, (S_, S_), 1)
    u_strict = jnp.where(rid < cid, 1.0, 0.0).astype(jnp.float32)

    for b in range(B_):
        o_b = o_ref[b]
        m_b = m_ref[b]
        dot = jax.lax.dot_general(
            o_b, m_b, (((1,), (1,)), ((), ())),
            preferred_element_type=jnp.float32,
        )
        sq_o = jnp.sum(o_b * o_b, axis=1)[:, None]
        sq_m = jnp.sum(m_b * m_b, axis=1)[None, :]
        sum_o = jnp.sum(o_b, axis=1)[:, None]
        sum_m = jnp.sum(m_b, axis=1)[None, :]
        dist2 = (sq_o + sq_m - 2.0 * dot
                 + (2.0 * EPS_) * (sum_o - sum_m)
                 + D_ * EPS_ * EPS_)
        cost = jnp.sqrt(jnp.maximum(dist2, 0.0))
        e = jax.lax.dot_general(
            cost, u_strict, (((1,), (0,)), ((), ())),
            preferred_element_type=jnp.float32,
        )
        c = e + cost
        c_ref[:, b, :] = c
        e_ref[:, b, :] = e

    prev0 = c_ref[0]

    def step(i, prev):
        c_i = c_ref[i]
        e_i = e_ref[i]
        t = prev - c_i
        w = prev - e_i
        mins = [jnp.min(t[:, c * CH_:(c + 1) * CH_], axis=1, keepdims=True)
                for c in range(NCH_ - 1)]
        outs = []
        m_pfx = None
        for c in range(NCH_):
            sl = slice(c * CH_, (c + 1) * CH_)
            tc = t[:, sl]
            a = _tree_min([tc] + [_shift_right(tc, s, POS_BIG)
                                  for s in range(1, RC_)])
            mm = _tree_min([_shift_right(a, 1 + RC_ * k, POS_BIG)
                            for k in range(RC_)
                            if 1 + RC_ * k < CH_])
            if c > 0:
                m_pfx = (mins[c - 1] if m_pfx is None
                         else jnp.minimum(m_pfx, mins[c - 1]))
                mm = jnp.minimum(mm, m_pfx)
            outs.append(c_i[:, sl] + jnp.minimum(w[:, sl], mm))
        return jnp.concatenate(outs, axis=1)

    last = jax.lax.fori_loop(1, S_, step, prev0)

    loss = jnp.sum(last[:, S_ - 1]) * (1.0 / B_)
    out_ref[...] = jnp.full((1, 1), loss, jnp.float32)


@jax.jit
def _dtw_loss(observed, modeled):
    out = pl.pallas_call(
        _dtw_body,
        out_shape=jax.ShapeDtypeStruct((1, 1), jnp.float32),
        in_specs=[
            pl.BlockSpec(memory_space=pltpu.VMEM),
            pl.BlockSpec(memory_space=pltpu.VMEM),
        ],
        out_specs=pl.BlockSpec(memory_space=pltpu.VMEM),
        scratch_shapes=[
            pltpu.VMEM((S_, B_, S_), jnp.float32),
            pltpu.VMEM((S_, B_, S_), jnp.float32),
        ],
    )(observed, modeled)
    return out[0, 0]


def kernel(observed, modeled):
    return _dtw_loss(observed, modeled)


# speedup vs baseline: 37.7422x; 1.1372x over previous
import jax
import jax.numpy as jnp
from jax.experimental import pallas as pl
from jax.experimental.pallas import tpu as pltpu

EPS_ = 1e-6
B_, S_, D_ = 8, 512, 64
POS_BIG = 3.0e38


def _shift_right(x, s, fill):
    return jnp.concatenate(
        [jnp.full((x.shape[0], s), fill, x.dtype), x[:, :-s]], axis=1
    )


def _tree_min(vals):
    while len(vals) > 1:
        nxt = [jnp.minimum(vals[i], vals[i + 1])
               for i in range(0, len(vals) - 1, 2)]
        if len(vals) % 2:
            nxt.append(vals[-1])
        vals = nxt
    return vals[0]


CH_ = 128
NCH_ = 4
RC_ = 12


def _local_scan(x):
    return _tree_min([x] + [_shift_right(x, s, POS_BIG) for s in range(1, R_)])


def _dtw_body(o_ref, m_ref, out_ref, c_ref, e_ref):
    rid = jax.lax.broadcasted_iota(jnp.int32, (S_, S_), 0)
    cid = jax.lax.broadcasted_iota(jnp.int32, (S_, S_), 1)
    u_strict = jnp.where(rid < cid, 1.0, 0.0).astype(jnp.float32)

    for b in range(B_):
        o_b = o_ref[b]
        m_b = m_ref[b]
        dot = jax.lax.dot_general(
            o_b, m_b, (((1,), (1,)), ((), ())),
            preferred_element_type=jnp.float32,
        )
        sq_o = jnp.sum(o_b * o_b, axis=1)[:, None]
        sq_m = jnp.sum(m_b * m_b, axis=1)[None, :]
        sum_o = jnp.sum(o_b, axis=1)[:, None]
        sum_m = jnp.sum(m_b, axis=1)[None, :]
        dist2 = (sq_o + sq_m - 2.0 * dot
                 + (2.0 * EPS_) * (sum_o - sum_m)
                 + D_ * EPS_ * EPS_)
        cost = jnp.sqrt(jnp.maximum(dist2, 0.0))
        e = jax.lax.dot_general(
            cost, u_strict, (((1,), (0,)), ((), ())),
            preferred_element_type=jnp.float32,
        )
        c = e + cost
        c_ref[:, b, :] = c
        e_ref[:, b, :] = e

    prev0 = c_ref[0]

    def row(i, prev_chunks):
        c_i = c_ref[i]
        e_i = e_ref[i]
        outs = []
        m_pfx = None
        for c in range(NCH_):
            sl = slice(c * CH_, (c + 1) * CH_)
            tc = prev_chunks[c] - c_i[:, sl]
            wc = prev_chunks[c] - e_i[:, sl]
            a = _tree_min([tc] + [_shift_right(tc, s, POS_BIG)
                                  for s in range(1, RC_)])
            parts = [wc] + [_shift_right(a, 1 + RC_ * k, POS_BIG)
                            for k in range(RC_)
                            if 1 + RC_ * k < CH_]
            if c > 0:
                cm = a_prev[:, CH_ - 1:CH_]
                m_pfx = cm if m_pfx is None else jnp.minimum(m_pfx, cm)
                parts.append(jnp.broadcast_to(m_pfx, (B_, CH_)))
            a_prev = a
            outs.append(c_i[:, sl] + _tree_min(parts))
        return tuple(outs)

    chunks0 = tuple(prev0[:, c * CH_:(c + 1) * CH_] for c in range(NCH_))

    def step2(k, carry):
        carry = row(2 * k + 1, carry)
        return row(2 * k + 2, carry)

    chunks = jax.lax.fori_loop(0, (S_ - 1) // 2, step2, chunks0)
    chunks = row(S_ - 1, chunks)
    last = chunks[NCH_ - 1]

    loss = jnp.sum(last[:, CH_ - 1]) * (1.0 / B_)
    out_ref[...] = jnp.full((1, 1), loss, jnp.float32)


@jax.jit
def _dtw_loss(observed, modeled):
    out = pl.pallas_call(
        _dtw_body,
        out_shape=jax.ShapeDtypeStruct((1, 1), jnp.float32),
        in_specs=[
            pl.BlockSpec(memory_space=pltpu.VMEM),
            pl.BlockSpec(memory_space=pltpu.VMEM),
        ],
        out_specs=pl.BlockSpec(memory_space=pltpu.VMEM),
        scratch_shapes=[
            pltpu.VMEM((S_, B_, S_), jnp.float32),
            pltpu.VMEM((S_, B_, S_), jnp.float32),
        ],
    )(observed, modeled)
    return out[0, 0]


def kernel(observed, modeled):
    return _dtw_loss(observed, modeled)


# speedup vs baseline: 37.7510x; 1.0002x over previous
import jax
import jax.numpy as jnp
from jax.experimental import pallas as pl
from jax.experimental.pallas import tpu as pltpu

EPS_ = 1e-6
B_, S_, D_ = 8, 512, 64
POS_BIG = 3.0e38


def _shift_right(x, s, fill):
    return jnp.concatenate(
        [jnp.full((x.shape[0], s), fill, x.dtype), x[:, :-s]], axis=1
    )


def _tree_min(vals):
    while len(vals) > 1:
        nxt = [jnp.minimum(vals[i], vals[i + 1])
               for i in range(0, len(vals) - 1, 2)]
        if len(vals) % 2:
            nxt.append(vals[-1])
        vals = nxt
    return vals[0]


CH_ = 128
NCH_ = 4
RC_ = 12


def _local_scan(x):
    return _tree_min([x] + [_shift_right(x, s, POS_BIG) for s in range(1, R_)])


def _dtw_body(o_ref, m_ref, out_ref, c_ref, e_ref):
    rid = jax.lax.broadcasted_iota(jnp.int32, (S_, S_), 0)
    cid = jax.lax.broadcasted_iota(jnp.int32, (S_, S_), 1)
    u_strict = jnp.where(rid < cid, 1.0, 0.0).astype(jnp.float32)

    for b in range(B_):
        o_b = o_ref[b]
        m_b = m_ref[b]
        dot = jax.lax.dot_general(
            o_b, m_b, (((1,), (1,)), ((), ())),
            preferred_element_type=jnp.float32,
        )
        sq_o = jnp.sum(o_b * o_b, axis=1)[:, None]
        sq_m = jnp.sum(m_b * m_b, axis=1)[None, :]
        sum_o = jnp.sum(o_b, axis=1)[:, None]
        sum_m = jnp.sum(m_b, axis=1)[None, :]
        dist2 = (sq_o + sq_m - 2.0 * dot
                 + (2.0 * EPS_) * (sum_o - sum_m)
                 + D_ * EPS_ * EPS_)
        cost = jnp.sqrt(jnp.maximum(dist2, 0.0))
        e = jax.lax.dot_general(
            cost, u_strict, (((1,), (0,)), ((), ())),
            preferred_element_type=jnp.float32,
        )
        c = e + cost
        c_ref[:, b, :] = c
        e_ref[:, b, :] = e

    prev0 = c_ref[0]

    def row(i, prev_chunks):
        c_i = c_ref[i]
        e_i = e_ref[i]
        outs = []
        m_pfx = None
        for c in range(NCH_):
            sl = slice(c * CH_, (c + 1) * CH_)
            tc = prev_chunks[c] - c_i[:, sl]
            wc = prev_chunks[c] - e_i[:, sl]
            a = _tree_min([tc] + [_shift_right(tc, s, POS_BIG)
                                  for s in range(1, RC_)])
            parts = [wc] + [_shift_right(a, 1 + RC_ * k, POS_BIG)
                            for k in range(RC_)
                            if 1 + RC_ * k < CH_]
            if c > 0:
                cm = a_prev[:, CH_ - 1:CH_]
                m_pfx = cm if m_pfx is None else jnp.minimum(m_pfx, cm)
                parts.append(jnp.broadcast_to(m_pfx, (B_, CH_)))
            a_prev = a
            outs.append(c_i[:, sl] + _tree_min(parts))
        return tuple(outs)

    chunks0 = tuple(prev0[:, c * CH_:(c + 1) * CH_] for c in range(NCH_))

    UN = 2

    def stepu(k, carry):
        for u in range(UN):
            carry = row(UN * k + 1 + u, carry)
        return carry

    n_unrolled = (S_ - 1) // UN
    chunks = jax.lax.fori_loop(0, n_unrolled, stepu, chunks0)
    for i in range(UN * n_unrolled + 1, S_):
        chunks = row(i, chunks)
    last = chunks[NCH_ - 1]

    loss = jnp.sum(last[:, CH_ - 1]) * (1.0 / B_)
    out_ref[...] = jnp.full((1, 1), loss, jnp.float32)


@jax.jit
def _dtw_loss(observed, modeled):
    out = pl.pallas_call(
        _dtw_body,
        out_shape=jax.ShapeDtypeStruct((1, 1), jnp.float32),
        in_specs=[
            pl.BlockSpec(memory_space=pltpu.VMEM),
            pl.BlockSpec(memory_space=pltpu.VMEM),
        ],
        out_specs=pl.BlockSpec(memory_space=pltpu.VMEM),
        scratch_shapes=[
            pltpu.VMEM((S_, B_, S_), jnp.float32),
            pltpu.VMEM((S_, B_, S_), jnp.float32),
        ],
    )(observed, modeled)
    return out[0, 0]


def kernel(observed, modeled):
    return _dtw_loss(observed, modeled)
